# baseline (speedup 1.0000x reference)
import functools

import jax
import jax.numpy as jnp
from jax import lax
from jax.experimental import pallas as pl
from jax.experimental.pallas import tpu as pltpu
from jax.experimental.pallas import tpu_sc as plsc

NC = 2
NS = 16
LANES = 16
K_EDGE = 128


def _gelu(h):
    return 0.5 * h * (1.0 + lax.erf(h * 0.7071067811865476))


def _layernorm(h, scale, bias, eps=1e-5):
    mu = jnp.mean(h, axis=-1, keepdims=True)
    var = jnp.mean((h - mu) ** 2, axis=-1, keepdims=True)
    return (h - mu) * jax.lax.rsqrt(var + eps) * scale + bias


def _gidx_body(src_ref, et_ref, o_ref, *, n_nodes, rn):
    base = et_ref[0, :] * n_nodes + src_ref[0, :]
    o_ref[0, :] = base
    o_ref[1, :] = base + rn


def _xw_body(x_ref, w_ref, o_ref, *, dh):
    y = jnp.dot(x_ref[...], w_ref[0], preferred_element_type=jnp.float32)
    o_ref[0, 0] = y[:, :dh]
    o_ref[1, 0] = y[:, dh:]


def _block_body(x_ref, lo_ref, hi_ref, lw_ref, cb_ref, pns_ref, pnb_ref,
                fns_ref, fnb_ref, w1_ref, b1_ref, w2_ref, b2_ref, o_ref):
    x = x_ref[...]
    agg = jnp.concatenate([lo_ref[...], hi_ref[...]], axis=1)
    h = agg + jnp.dot(x, lw_ref[...], preferred_element_type=jnp.float32)
    h = h + cb_ref[...]
    h = _layernorm(h, pns_ref[...], pnb_ref[...])
    h = _gelu(h)
    x1 = x + h
    f = _layernorm(x1, fns_ref[...], fnb_ref[...])
    f = jnp.dot(f, w1_ref[...], preferred_element_type=jnp.float32) + b1_ref[...]
    f = _gelu(f)
    f = jnp.dot(f, w2_ref[...], preferred_element_type=jnp.float32) + b2_ref[...]
    o_ref[...] = x1 + f


def kernel(graph, x, edge_type, W_rel, loop_weight, conv_bias,
           pre_norm_scale, pre_norm_bias, ff_norm_scale, ff_norm_bias,
           W1, b1, W2, b2):
    n_nodes, d = x.shape
    r_rel = W_rel.shape[0]
    n_edges = edge_type.shape[0]
    hff = W1.shape[1]
    dh = d // 2

    grain = NS * K_EDGE * 4
    e_pad = ((n_edges + grain - 1) // grain) * grain
    pad = e_pad - n_edges
    src = graph[0].astype(jnp.int32)
    dst = graph[1].astype(jnp.int32)
    et = edge_type.astype(jnp.int32)
    if pad:
        src = jnp.concatenate([src, jnp.zeros((pad,), jnp.int32)])
        dst = jnp.concatenate([dst, jnp.full((pad,), n_nodes, jnp.int32)])
        et = jnp.concatenate([et, jnp.zeros((pad,), jnp.int32)])
    dst2 = dst.reshape(e_pad // K_EDGE, K_EDGE)

    n_acc = ((n_nodes + 1 + NS * 128 - 1) // (NS * 128)) * (NS * 128)

    bn1 = 1000
    nb1 = n_nodes // bn1
    xw = pl.pallas_call(
        functools.partial(_xw_body, dh=dh),
        grid=(nb1, r_rel),
        in_specs=[
            pl.BlockSpec((bn1, d), lambda i, r: (i, 0)),
            pl.BlockSpec((1, d, d), lambda i, r: (r, 0, 0)),
        ],
        out_specs=pl.BlockSpec((2, 1, bn1, dh), lambda i, r: (0, r, i, 0)),
        out_shape=jax.ShapeDtypeStruct((2, r_rel, n_nodes, dh), jnp.float32),
    )(x, W_rel)
    table = xw.reshape(2 * r_rel * n_nodes, dh)

    rows_per_tile = n_acc // NS
    ept = e_pad // NS
    nchunks = ept // K_EDGE
    rn = r_rel * n_nodes

    bg = 16384
    g2 = pl.pallas_call(
        functools.partial(_gidx_body, n_nodes=n_nodes, rn=rn),
        grid=(e_pad // bg,),
        in_specs=[
            pl.BlockSpec((1, bg), lambda i: (0, i)),
            pl.BlockSpec((1, bg), lambda i: (0, i)),
        ],
        out_specs=pl.BlockSpec((2, bg), lambda i: (0, i)),
        out_shape=jax.ShapeDtypeStruct((2, e_pad), jnp.int32),
    )(src.reshape(1, e_pad), et.reshape(1, e_pad))
    g2f = g2.reshape(2 * e_pad)

    npairs = nchunks // 2
    nquads = npairs // 2
    kp = 2 * K_EDGE

    def _sc_body(table_ref, g2_ref, dst2_ref, lo_ref, hi_ref,
                 ga_v, gb_v, dst_v, rows0_v, rows1_v,
                 acc_sh, sem0, sem1):
        c = lax.axis_index("c")
        s = lax.axis_index("s")
        tbase = s * rows_per_tile

        def _zrow(i, carry):
            for j in range(dh // LANES):
                rows0_v[i, pl.ds(j * LANES, LANES)] = jnp.zeros((LANES,), jnp.float32)
            return carry
        lax.fori_loop(0, K_EDGE, _zrow, 0)
        for k in range(rows_per_tile // K_EDGE):
            pltpu.sync_copy(rows0_v, acc_sh.at[pl.ds(tbase + k * K_EDGE, K_EDGE)])
        plsc.subcore_barrier()

        gbase0 = c * e_pad + s * ept
        drow0 = s * nchunks

        def _stage(pi, gbuf, dbase):
            gb = pl.multiple_of(gbase0 + pi * kp, kp)
            pltpu.sync_copy(g2_ref.at[pl.ds(gb, kp)], gbuf)
            dr = pl.multiple_of(drow0 + pi * 2, 2)
            pltpu.sync_copy(dst2_ref.at[pl.ds(dr, 2)], dst_v.at[pl.ds(dbase, 2)])

        def _start(gbuf, leg, buf, sem):
            sl = pl.ds(leg * K_EDGE, K_EDGE)
            return pltpu.async_copy(table_ref.at[gbuf.at[sl]], buf, sem)

        def _scat(buf, di):
            pltpu.sync_copy(buf, acc_sh.at[dst_v.at[di]], add=True)

        _stage(0, ga_v, 0)

        def _quad(i, carry):
            ha0 = _start(ga_v, 0, rows0_v, sem0)
            ha1 = _start(ga_v, 1, rows1_v, sem1)
            _stage(2 * i + 1, gb_v, 2)
            ha0.wait()
            _scat(rows0_v, 0)
            hb0 = _start(gb_v, 0, rows0_v, sem0)
            ha1.wait()
            _scat(rows1_v, 1)
            hb1 = _start(gb_v, 1, rows1_v, sem1)

            @pl.when(i < nquads - 1)
            def _():
                _stage(2 * i + 2, ga_v, 0)
            hb0.wait()
            _scat(rows0_v, 2)
            hb1.wait()
            _scat(rows1_v, 3)
            return carry
        lax.fori_loop(0, nquads, _quad, 0)
        plsc.subcore_barrier()

        def _copy_out(out_ref):
            for k in range(rows_per_tile // K_EDGE):
                sl = pl.ds(tbase + k * K_EDGE, K_EDGE)
                pltpu.sync_copy(acc_sh.at[sl], rows0_v)
                pltpu.sync_copy(rows0_v, out_ref.at[sl])

        @pl.when(c == 0)
        def _():
            _copy_out(lo_ref)

        @pl.when(c == 1)
        def _():
            _copy_out(hi_ref)

    lo, hi = pl.kernel(
        _sc_body,
        out_type=[
            jax.ShapeDtypeStruct((n_acc, dh), jnp.float32),
            jax.ShapeDtypeStruct((n_acc, dh), jnp.float32),
        ],
        mesh=plsc.VectorSubcoreMesh(core_axis_name="c", subcore_axis_name="s"),
        scratch_types=[
            pltpu.VMEM((kp,), jnp.int32),
            pltpu.VMEM((kp,), jnp.int32),
            pltpu.VMEM((4, K_EDGE), jnp.int32),
            pltpu.VMEM((K_EDGE, dh), jnp.float32),
            pltpu.VMEM((K_EDGE, dh), jnp.float32),
            pltpu.VMEM_SHARED((n_acc, dh), jnp.float32),
            pltpu.SemaphoreType.DMA,
            pltpu.SemaphoreType.DMA,
        ],
    )(table, g2f, dst2)

    bn2 = 1000
    nb2 = n_nodes // bn2
    cb = conv_bias.reshape(1, d)
    pns = pre_norm_scale.reshape(1, d)
    pnb = pre_norm_bias.reshape(1, d)
    fns = ff_norm_scale.reshape(1, d)
    fnb = ff_norm_bias.reshape(1, d)
    b1r = b1.reshape(1, hff)
    b2r = b2.reshape(1, d)
    out = pl.pallas_call(
        _block_body,
        grid=(nb2,),
        in_specs=[
            pl.BlockSpec((bn2, d), lambda i: (i, 0)),
            pl.BlockSpec((bn2, dh), lambda i: (i, 0)),
            pl.BlockSpec((bn2, dh), lambda i: (i, 0)),
            pl.BlockSpec((d, d), lambda i: (0, 0)),
            pl.BlockSpec((1, d), lambda i: (0, 0)),
            pl.BlockSpec((1, d), lambda i: (0, 0)),
            pl.BlockSpec((1, d), lambda i: (0, 0)),
            pl.BlockSpec((1, d), lambda i: (0, 0)),
            pl.BlockSpec((1, d), lambda i: (0, 0)),
            pl.BlockSpec((d, hff), lambda i: (0, 0)),
            pl.BlockSpec((1, hff), lambda i: (0, 0)),
            pl.BlockSpec((hff, d), lambda i: (0, 0)),
            pl.BlockSpec((1, d), lambda i: (0, 0)),
        ],
        out_specs=pl.BlockSpec((bn2, d), lambda i: (i, 0)),
        out_shape=jax.ShapeDtypeStruct((n_nodes, d), jnp.float32),
    )(x, lo, hi, loop_weight, cb, pns, pnb, fns, fnb, W1, b1r, W2, b2r)
    return out


# speedup vs baseline: 1.0088x; 1.0088x over previous
import functools

import jax
import jax.numpy as jnp
from jax import lax
from jax.experimental import pallas as pl
from jax.experimental.pallas import tpu as pltpu
from jax.experimental.pallas import tpu_sc as plsc

NC = 2
NS = 16
LANES = 16
K_EDGE = 128


def _gelu(h):
    return 0.5 * h * (1.0 + lax.erf(h * 0.7071067811865476))


def _layernorm(h, scale, bias, eps=1e-5):
    mu = jnp.mean(h, axis=-1, keepdims=True)
    var = jnp.mean((h - mu) ** 2, axis=-1, keepdims=True)
    return (h - mu) * jax.lax.rsqrt(var + eps) * scale + bias


def _gidx_body(src_ref, et_ref, o_ref, *, n_nodes, rn):
    base = et_ref[0, :] * n_nodes + src_ref[0, :]
    o_ref[0, :] = base
    o_ref[1, :] = base + rn


def _xw_body(x_ref, w_ref, o_ref, *, dh):
    y = jnp.dot(x_ref[...], w_ref[0], preferred_element_type=jnp.float32)
    o_ref[0, 0] = y[:, :dh]
    o_ref[1, 0] = y[:, dh:]


def _block_body(x_ref, lo_ref, hi_ref, lw_ref, cb_ref, pns_ref, pnb_ref,
                fns_ref, fnb_ref, w1_ref, b1_ref, w2_ref, b2_ref, o_ref):
    x = x_ref[...]
    agg = jnp.concatenate([lo_ref[...], hi_ref[...]], axis=1)
    h = agg + jnp.dot(x, lw_ref[...], preferred_element_type=jnp.float32)
    h = h + cb_ref[...]
    h = _layernorm(h, pns_ref[...], pnb_ref[...])
    h = _gelu(h)
    x1 = x + h
    f = _layernorm(x1, fns_ref[...], fnb_ref[...])
    f = jnp.dot(f, w1_ref[...], preferred_element_type=jnp.float32) + b1_ref[...]
    f = _gelu(f)
    f = jnp.dot(f, w2_ref[...], preferred_element_type=jnp.float32) + b2_ref[...]
    o_ref[...] = x1 + f


def kernel(graph, x, edge_type, W_rel, loop_weight, conv_bias,
           pre_norm_scale, pre_norm_bias, ff_norm_scale, ff_norm_bias,
           W1, b1, W2, b2):
    n_nodes, d = x.shape
    r_rel = W_rel.shape[0]
    n_edges = edge_type.shape[0]
    hff = W1.shape[1]
    dh = d // 2

    grain = NS * K_EDGE * 4
    e_pad = ((n_edges + grain - 1) // grain) * grain
    pad = e_pad - n_edges
    src = graph[0].astype(jnp.int32)
    dst = graph[1].astype(jnp.int32)
    et = edge_type.astype(jnp.int32)
    if pad:
        src = jnp.concatenate([src, jnp.zeros((pad,), jnp.int32)])
        dst = jnp.concatenate([dst, jnp.full((pad,), n_nodes, jnp.int32)])
        et = jnp.concatenate([et, jnp.zeros((pad,), jnp.int32)])
    dst2 = dst.reshape(e_pad // K_EDGE, K_EDGE)

    n_acc = ((n_nodes + 1 + NS * 128 - 1) // (NS * 128)) * (NS * 128)

    bn1 = 1000
    nb1 = n_nodes // bn1
    xw = pl.pallas_call(
        functools.partial(_xw_body, dh=dh),
        grid=(nb1, r_rel),
        in_specs=[
            pl.BlockSpec((bn1, d), lambda i, r: (i, 0)),
            pl.BlockSpec((1, d, d), lambda i, r: (r, 0, 0)),
        ],
        out_specs=pl.BlockSpec((2, 1, bn1, dh), lambda i, r: (0, r, i, 0)),
        out_shape=jax.ShapeDtypeStruct((2, r_rel, n_nodes, dh), jnp.float32),
    )(x, W_rel)
    table = xw.reshape(2 * r_rel * n_nodes, dh)

    rows_per_tile = n_acc // NS
    ept = e_pad // NS
    nchunks = ept // K_EDGE
    rn = r_rel * n_nodes

    bg = 16384
    g2 = pl.pallas_call(
        functools.partial(_gidx_body, n_nodes=n_nodes, rn=rn),
        grid=(e_pad // bg,),
        in_specs=[
            pl.BlockSpec((1, bg), lambda i: (0, i)),
            pl.BlockSpec((1, bg), lambda i: (0, i)),
        ],
        out_specs=pl.BlockSpec((2, bg), lambda i: (0, i)),
        out_shape=jax.ShapeDtypeStruct((2, e_pad), jnp.int32),
    )(src.reshape(1, e_pad), et.reshape(1, e_pad))
    g2f = g2.reshape(2 * e_pad)

    npairs = nchunks // 2
    nquads = npairs // 2
    kp = 2 * K_EDGE

    def _sc_body(table_ref, g2_ref, dst2_ref, lo_ref, hi_ref,
                 ga_v, gb_v, dst_v, rows0_v, rows1_v,
                 acc_sh, sem0, sem1):
        c = lax.axis_index("c")
        s = lax.axis_index("s")
        tbase = s * rows_per_tile

        def _zrow(i, carry):
            for j in range(dh // LANES):
                rows0_v[i, pl.ds(j * LANES, LANES)] = jnp.zeros((LANES,), jnp.float32)
            return carry
        lax.fori_loop(0, K_EDGE, _zrow, 0)
        for k in range(rows_per_tile // K_EDGE):
            pltpu.sync_copy(rows0_v, acc_sh.at[pl.ds(tbase + k * K_EDGE, K_EDGE)])
        plsc.subcore_barrier()

        gbase0 = c * e_pad + s * ept
        drow0 = s * nchunks

        pltpu.sync_copy(dst2_ref.at[pl.ds(drow0, nchunks)], dst_v)

        def _stage(pi, gbuf):
            gb = pl.multiple_of(gbase0 + pi * kp, kp)
            pltpu.sync_copy(g2_ref.at[pl.ds(gb, kp)], gbuf)

        def _start(gbuf, leg, buf, sem):
            sl = pl.ds(leg * K_EDGE, K_EDGE)
            return pltpu.async_copy(table_ref.at[gbuf.at[sl]], buf, sem)

        def _scat(buf, ci):
            pltpu.sync_copy(buf, acc_sh.at[dst_v.at[ci]], add=True)

        _stage(0, ga_v)

        def _quad(i, carry):
            ha0 = _start(ga_v, 0, rows0_v, sem0)
            ha1 = _start(ga_v, 1, rows1_v, sem1)
            _stage(2 * i + 1, gb_v)
            ha0.wait()
            _scat(rows0_v, 4 * i)
            hb0 = _start(gb_v, 0, rows0_v, sem0)
            ha1.wait()
            _scat(rows1_v, 4 * i + 1)
            hb1 = _start(gb_v, 1, rows1_v, sem1)

            @pl.when(i < nquads - 1)
            def _():
                _stage(2 * i + 2, ga_v)
            hb0.wait()
            _scat(rows0_v, 4 * i + 2)
            hb1.wait()
            _scat(rows1_v, 4 * i + 3)
            return carry
        lax.fori_loop(0, nquads, _quad, 0)
        plsc.subcore_barrier()

        def _copy_out(out_ref):
            for k in range(rows_per_tile // K_EDGE):
                sl = pl.ds(tbase + k * K_EDGE, K_EDGE)
                pltpu.sync_copy(acc_sh.at[sl], rows0_v)
                pltpu.sync_copy(rows0_v, out_ref.at[sl])

        @pl.when(c == 0)
        def _():
            _copy_out(lo_ref)

        @pl.when(c == 1)
        def _():
            _copy_out(hi_ref)

    lo, hi = pl.kernel(
        _sc_body,
        out_type=[
            jax.ShapeDtypeStruct((n_acc, dh), jnp.float32),
            jax.ShapeDtypeStruct((n_acc, dh), jnp.float32),
        ],
        mesh=plsc.VectorSubcoreMesh(core_axis_name="c", subcore_axis_name="s"),
        scratch_types=[
            pltpu.VMEM((kp,), jnp.int32),
            pltpu.VMEM((kp,), jnp.int32),
            pltpu.VMEM((nchunks, K_EDGE), jnp.int32),
            pltpu.VMEM((K_EDGE, dh), jnp.float32),
            pltpu.VMEM((K_EDGE, dh), jnp.float32),
            pltpu.VMEM_SHARED((n_acc, dh), jnp.float32),
            pltpu.SemaphoreType.DMA,
            pltpu.SemaphoreType.DMA,
        ],
    )(table, g2f, dst2)

    bn2 = 1000
    nb2 = n_nodes // bn2
    cb = conv_bias.reshape(1, d)
    pns = pre_norm_scale.reshape(1, d)
    pnb = pre_norm_bias.reshape(1, d)
    fns = ff_norm_scale.reshape(1, d)
    fnb = ff_norm_bias.reshape(1, d)
    b1r = b1.reshape(1, hff)
    b2r = b2.reshape(1, d)
    out = pl.pallas_call(
        _block_body,
        grid=(nb2,),
        in_specs=[
            pl.BlockSpec((bn2, d), lambda i: (i, 0)),
            pl.BlockSpec((bn2, dh), lambda i: (i, 0)),
            pl.BlockSpec((bn2, dh), lambda i: (i, 0)),
            pl.BlockSpec((d, d), lambda i: (0, 0)),
            pl.BlockSpec((1, d), lambda i: (0, 0)),
            pl.BlockSpec((1, d), lambda i: (0, 0)),
            pl.BlockSpec((1, d), lambda i: (0, 0)),
            pl.BlockSpec((1, d), lambda i: (0, 0)),
            pl.BlockSpec((1, d), lambda i: (0, 0)),
            pl.BlockSpec((d, hff), lambda i: (0, 0)),
            pl.BlockSpec((1, hff), lambda i: (0, 0)),
            pl.BlockSpec((hff, d), lambda i: (0, 0)),
            pl.BlockSpec((1, d), lambda i: (0, 0)),
        ],
        out_specs=pl.BlockSpec((bn2, d), lambda i: (i, 0)),
        out_shape=jax.ShapeDtypeStruct((n_nodes, d), jnp.float32),
    )(x, lo, hi, loop_weight, cb, pns, pnb, fns, fnb, W1, b1r, W2, b2r)
    return out
